# baseline (speedup 1.0000x reference)
import jax
import jax.numpy as jnp
from jax import lax
from jax.experimental import pallas as pl
from jax.experimental.pallas import tpu as pltpu
from jax.experimental.pallas import tpu_sc as plsc

D_MODEL = 64
SCALE = 8.0
LANES = 16
PADW = 128

NUM_CORES = 2
NUM_SUBCORES = 16
NUM_WORKERS = NUM_CORES * NUM_SUBCORES

SEQ = 200
BATCH = 4096
BW = BATCH // NUM_WORKERS
SBLK = 8
NBUF = 4
VB = 2048


def _pad_body(tt_ref, out_ref):
    out_ref[:, pl.ds(0, D_MODEL)] = tt_ref[...].T


def _emb_body(xt_hbm, padded_hbm, out_hbm, idx_v, rows, trans,
              gsems, wsems):
    wid = lax.axis_index("s") * NUM_CORES + lax.axis_index("c")
    b0 = wid * BW

    def stage(q, slot):
        pltpu.sync_copy(xt_hbm.at[pl.ds(q * SBLK, SBLK), pl.ds(b0, BW)],
                        idx_v.at[pl.ds(slot * SBLK, SBLK), :])

    def start_gather(c, b):
        row = (c // SBLK) % 2 * SBLK + c % SBLK
        pltpu.async_copy(padded_hbm.at[idx_v.at[row]], rows[b], gsems[b])

    stage(0, 0)
    for b in range(NBUF):
        start_gather(b, b)

    def process(c, b):
        @pl.when(jnp.logical_and(c % SBLK == 0, c // SBLK + 1 < SEQ // SBLK))
        def _():
            stage(c // SBLK + 1, (c // SBLK + 1) % 2)

        pltpu.make_async_copy(padded_hbm.at[idx_v.at[c]], rows[b],
                              gsems[b]).wait()

        @pl.when(c >= NBUF)
        def _():
            pltpu.make_async_copy(
                trans[b], out_hbm.at[c, :, pl.ds(b0, BW)], wsems[b]).wait()

        irow = (c // SBLK) % 2 * SBLK + c % SBLK

        def group(g, carry):
            idxvec = idx_v[irow, pl.ds(g * LANES, LANES)]
            svec = jnp.where(idxvec == 0, 0.0, SCALE).astype(jnp.float32)
            lanes = lax.iota(jnp.int32, LANES)
            rowvec = g * LANES + lanes
            for k in range(D_MODEL):
                dvec = (lanes + k) & (D_MODEL - 1)
                val = plsc.load_gather(rows[b], [rowvec, dvec])
                plsc.store_scatter(trans[b], [dvec, rowvec], val * svec)
            return carry

        lax.fori_loop(0, BW // LANES, group, 0)
        pltpu.async_copy(trans[b], out_hbm.at[c, :, pl.ds(b0, BW)], wsems[b])

        @pl.when(c + NBUF < SEQ)
        def _():
            start_gather(c + NBUF, b)

    def outer(g, carry):
        for b in range(NBUF):
            process(g * NBUF + b, b)
        return carry

    lax.fori_loop(0, SEQ // NBUF, outer, 0)
    for b in range(NBUF):
        c = SEQ - NBUF + b
        pltpu.make_async_copy(
            trans[b], out_hbm.at[c, :, pl.ds(b0, BW)], wsems[b]).wait()


def kernel(x, table):
    bsz, seq = x.shape
    vocab = table.shape[0]
    xt = x.T
    table_t = table.T

    padded = pl.pallas_call(
        _pad_body,
        grid=(pl.cdiv(vocab, VB),),
        in_specs=[pl.BlockSpec((D_MODEL, VB), lambda i: (0, i))],
        out_specs=pl.BlockSpec((VB, PADW), lambda i: (i, 0)),
        out_shape=jax.ShapeDtypeStruct((vocab, PADW), jnp.float32),
        compiler_params=pltpu.CompilerParams(
            dimension_semantics=("arbitrary",)),
    )(table_t)

    k = pl.kernel(
        _emb_body,
        out_type=jax.ShapeDtypeStruct((seq, D_MODEL, bsz), jnp.float32),
        mesh=plsc.VectorSubcoreMesh(
            core_axis_name="c", subcore_axis_name="s"),
        scratch_types=[
            pltpu.VMEM((2 * SBLK, BW), jnp.int32),
            [pltpu.VMEM((BW, PADW), jnp.float32) for _ in range(NBUF)],
            [pltpu.VMEM((D_MODEL, BW), jnp.float32) for _ in range(NBUF)],
            [pltpu.SemaphoreType.DMA for _ in range(NBUF)],
            [pltpu.SemaphoreType.DMA for _ in range(NBUF)],
        ],
        compiler_params=pltpu.CompilerParams(
            use_tc_tiling_on_sc=True, needs_layout_passes=False),
    )
    out_t = k(xt, padded)
    return jnp.transpose(out_t, (2, 0, 1))


# speedup vs baseline: 2.2022x; 2.2022x over previous
import jax
import jax.numpy as jnp
from jax import lax
from jax.experimental import pallas as pl
from jax.experimental.pallas import tpu as pltpu
from jax.experimental.pallas import tpu_sc as plsc

D_MODEL = 64
SCALE = 8.0
LANES = 16
PADW = 128

NUM_CORES = 2
NUM_SUBCORES = 16
NUM_WORKERS = NUM_CORES * NUM_SUBCORES

SEQ = 200
BATCH = 4096
BW = BATCH // NUM_WORKERS
SBLK = 8
NBUF = 4
VB = 8192


def _pad_body(tt_ref, out_ref):
    out_ref[:, pl.ds(0, D_MODEL)] = tt_ref[...].T


def _emb_body(xt_hbm, padded_hbm, out_hbm, idx_v, rows, trans,
              gsems, wsems):
    wid = lax.axis_index("s") * NUM_CORES + lax.axis_index("c")
    b0 = wid * BW

    def stage(q, slot):
        pltpu.sync_copy(xt_hbm.at[pl.ds(q * SBLK, SBLK), pl.ds(b0, BW)],
                        idx_v.at[pl.ds(slot * SBLK, SBLK), :])

    def start_gather(c, b):
        row = (c // SBLK) % 2 * SBLK + c % SBLK
        pltpu.async_copy(padded_hbm.at[idx_v.at[row]], rows[b], gsems[b])

    stage(0, 0)
    for b in range(NBUF):
        start_gather(b, b)

    def process(c, b):
        @pl.when(jnp.logical_and(c % SBLK == 0, c // SBLK + 1 < SEQ // SBLK))
        def _():
            stage(c // SBLK + 1, (c // SBLK + 1) % 2)

        pltpu.make_async_copy(padded_hbm.at[idx_v.at[c]], rows[b],
                              gsems[b]).wait()

        @pl.when(c >= NBUF)
        def _():
            pltpu.make_async_copy(
                trans[b], out_hbm.at[c, :, pl.ds(b0, BW)], wsems[b]).wait()

        irow = (c // SBLK) % 2 * SBLK + c % SBLK

        def group(g, carry):
            idxvec = idx_v[irow, pl.ds(g * LANES, LANES)]
            svec = jnp.where(idxvec == 0, 0.0, SCALE).astype(jnp.float32)
            lanes = lax.iota(jnp.int32, LANES)
            rowvec = g * LANES + lanes

            @plsc.parallel_loop(0, D_MODEL, step=1, unroll=8)
            def _(k):
                dvec = (lanes + k) & (D_MODEL - 1)
                val = plsc.load_gather(rows[b], [rowvec, dvec])
                plsc.store_scatter(trans[b], [dvec, rowvec], val * svec)

            return carry

        lax.fori_loop(0, BW // LANES, group, 0)
        pltpu.async_copy(trans[b], out_hbm.at[c, :, pl.ds(b0, BW)], wsems[b])

        @pl.when(c + NBUF < SEQ)
        def _():
            start_gather(c + NBUF, b)

    def outer(g, carry):
        for b in range(NBUF):
            process(g * NBUF + b, b)
        return carry

    lax.fori_loop(0, SEQ // NBUF, outer, 0)
    for b in range(NBUF):
        c = SEQ - NBUF + b
        pltpu.make_async_copy(
            trans[b], out_hbm.at[c, :, pl.ds(b0, BW)], wsems[b]).wait()


def kernel(x, table):
    bsz, seq = x.shape
    vocab = table.shape[0]
    xt = x.T
    table_t = table.T

    padded = pl.pallas_call(
        _pad_body,
        grid=(pl.cdiv(vocab, VB),),
        in_specs=[pl.BlockSpec((D_MODEL, VB), lambda i: (0, i))],
        out_specs=pl.BlockSpec((VB, PADW), lambda i: (i, 0)),
        out_shape=jax.ShapeDtypeStruct((vocab, PADW), jnp.float32),
        compiler_params=pltpu.CompilerParams(
            dimension_semantics=("arbitrary",)),
    )(table_t)

    k = pl.kernel(
        _emb_body,
        out_type=jax.ShapeDtypeStruct((seq, D_MODEL, bsz), jnp.float32),
        mesh=plsc.VectorSubcoreMesh(
            core_axis_name="c", subcore_axis_name="s"),
        scratch_types=[
            pltpu.VMEM((2 * SBLK, BW), jnp.int32),
            [pltpu.VMEM((BW, PADW), jnp.float32) for _ in range(NBUF)],
            [pltpu.VMEM((D_MODEL, BW), jnp.float32) for _ in range(NBUF)],
            [pltpu.SemaphoreType.DMA for _ in range(NBUF)],
            [pltpu.SemaphoreType.DMA for _ in range(NBUF)],
        ],
        compiler_params=pltpu.CompilerParams(
            use_tc_tiling_on_sc=True, needs_layout_passes=False),
    )
    out_t = k(xt, padded)
    return jnp.transpose(out_t, (2, 0, 1))


# speedup vs baseline: 2.2902x; 1.0400x over previous
import jax
import jax.numpy as jnp
from jax import lax
from jax.experimental import pallas as pl
from jax.experimental.pallas import tpu as pltpu
from jax.experimental.pallas import tpu_sc as plsc

D_MODEL = 64
SCALE = 8.0
LANES = 16
PADW = 128

NUM_CORES = 2
NUM_SUBCORES = 16
NUM_WORKERS = NUM_CORES * NUM_SUBCORES

SEQ = 200
BATCH = 4096
BW = BATCH // NUM_WORKERS
SBLK = 8
NBUF = 4
VB = 16384


def _pad_body(tt_ref, out_ref):
    out_ref[:, pl.ds(0, D_MODEL)] = tt_ref[...].T


def _emb_body(xt_hbm, padded_hbm, out_hbm, idx_v, rows, trans,
              gsems, wsems):
    wid = lax.axis_index("s") * NUM_CORES + lax.axis_index("c")
    b0 = wid * BW

    def stage(q, slot):
        pltpu.sync_copy(xt_hbm.at[pl.ds(q * SBLK, SBLK), pl.ds(b0, BW)],
                        idx_v.at[pl.ds(slot * SBLK, SBLK), :])

    def start_gather(c, b):
        row = (c // SBLK) % 2 * SBLK + c % SBLK
        pltpu.async_copy(padded_hbm.at[idx_v.at[row]], rows[b], gsems[b])

    stage(0, 0)
    for b in range(NBUF):
        start_gather(b, b)

    def process(c, b):
        @pl.when(jnp.logical_and(c % SBLK == 0, c // SBLK + 1 < SEQ // SBLK))
        def _():
            stage(c // SBLK + 1, (c // SBLK + 1) % 2)

        pltpu.make_async_copy(padded_hbm.at[idx_v.at[c]], rows[b],
                              gsems[b]).wait()

        @pl.when(c >= NBUF)
        def _():
            pltpu.make_async_copy(
                trans[b], out_hbm.at[c, :, pl.ds(b0, BW)], wsems[b]).wait()

        irow = (c // SBLK) % 2 * SBLK + c % SBLK

        def group(g, carry):
            idxvec = idx_v[irow, pl.ds(g * LANES, LANES)]
            svec = jnp.where(idxvec == 0, 0.0, SCALE).astype(jnp.float32)
            lanes = lax.iota(jnp.int32, LANES)
            rowvec = g * LANES + lanes

            @plsc.parallel_loop(0, D_MODEL, step=1, unroll=8)
            def _(k):
                dvec = (lanes + k) & (D_MODEL - 1)
                val = plsc.load_gather(rows[b], [rowvec, dvec])
                plsc.store_scatter(trans[b], [dvec, rowvec], val * svec)

            return carry

        lax.fori_loop(0, BW // LANES, group, 0)
        pltpu.async_copy(trans[b], out_hbm.at[c, :, pl.ds(b0, BW)], wsems[b])

        @pl.when(c + NBUF < SEQ)
        def _():
            start_gather(c + NBUF, b)

    def outer(g, carry):
        for b in range(NBUF):
            process(g * NBUF + b, b)
        return carry

    lax.fori_loop(0, SEQ // NBUF, outer, 0)
    for b in range(NBUF):
        c = SEQ - NBUF + b
        pltpu.make_async_copy(
            trans[b], out_hbm.at[c, :, pl.ds(b0, BW)], wsems[b]).wait()


def kernel(x, table):
    bsz, seq = x.shape
    vocab = table.shape[0]
    xt = x.T
    table_t = table.T

    padded = pl.pallas_call(
        _pad_body,
        grid=(pl.cdiv(vocab, VB),),
        in_specs=[pl.BlockSpec((D_MODEL, VB), lambda i: (0, i))],
        out_specs=pl.BlockSpec((VB, PADW), lambda i: (i, 0)),
        out_shape=jax.ShapeDtypeStruct((vocab, PADW), jnp.float32),
        compiler_params=pltpu.CompilerParams(
            dimension_semantics=("arbitrary",)),
    )(table_t)

    k = pl.kernel(
        _emb_body,
        out_type=jax.ShapeDtypeStruct((seq, D_MODEL, bsz), jnp.float32),
        mesh=plsc.VectorSubcoreMesh(
            core_axis_name="c", subcore_axis_name="s"),
        scratch_types=[
            pltpu.VMEM((2 * SBLK, BW), jnp.int32),
            [pltpu.VMEM((BW, PADW), jnp.float32) for _ in range(NBUF)],
            [pltpu.VMEM((D_MODEL, BW), jnp.float32) for _ in range(NBUF)],
            [pltpu.SemaphoreType.DMA for _ in range(NBUF)],
            [pltpu.SemaphoreType.DMA for _ in range(NBUF)],
        ],
        compiler_params=pltpu.CompilerParams(
            use_tc_tiling_on_sc=True, needs_layout_passes=False),
    )
    out_t = k(xt, padded)
    return jnp.transpose(out_t, (2, 0, 1))
